# baseline (device time: 58666 ns/iter reference)
import jax
import jax.numpy as jnp
from jax import lax
from jax.experimental import pallas as pl
from jax.experimental.pallas import tpu as pltpu


def kernel(partial, resid, gamma):
    M, D = resid.shape
    half = M // 2
    p2 = partial.reshape(M, D)
    g2 = gamma.reshape(1, D)

    def body(p_ref, r_ref, g_ref, o_ref, yrecv_ref, sems):
        my_x = lax.axis_index("x")
        my_y = lax.axis_index("y")
        row0 = my_x * half
        y_nbr = (my_x, 1 - my_y)
        x_nbr = (1 - my_x, my_y)

        barrier_sem = pltpu.get_barrier_semaphore()
        for nbr in (y_nbr, x_nbr):
            pl.semaphore_signal(
                barrier_sem, inc=1,
                device_id=nbr, device_id_type=pl.DeviceIdType.MESH,
            )
        pl.semaphore_wait(barrier_sem, 2)

        rdma_y = pltpu.make_async_remote_copy(
            src_ref=p_ref.at[pl.ds(row0, half), :],
            dst_ref=yrecv_ref,
            send_sem=sems.at[0],
            recv_sem=sems.at[1],
            device_id=y_nbr,
            device_id_type=pl.DeviceIdType.MESH,
        )
        rdma_y.start()
        rdma_y.wait()

        yh = p_ref[pl.ds(row0, half), :] + yrecv_ref[:, :] \
            + r_ref[pl.ds(row0, half), :]
        rms = jnp.sqrt(jnp.mean(yh * yh, axis=1, keepdims=True) + 1e-6)
        o_ref[pl.ds(row0, half), :] = yh / rms * g_ref[:, :]

        rdma_x = pltpu.make_async_remote_copy(
            src_ref=o_ref.at[pl.ds(row0, half), :],
            dst_ref=o_ref.at[pl.ds(row0, half), :],
            send_sem=sems.at[2],
            recv_sem=sems.at[3],
            device_id=x_nbr,
            device_id_type=pl.DeviceIdType.MESH,
        )
        rdma_x.start()
        rdma_x.wait()

    return pl.pallas_call(
        body,
        out_shape=jax.ShapeDtypeStruct((M, D), jnp.float32),
        in_specs=[
            pl.BlockSpec(memory_space=pltpu.VMEM),
            pl.BlockSpec(memory_space=pltpu.VMEM),
            pl.BlockSpec(memory_space=pltpu.VMEM),
        ],
        out_specs=pl.BlockSpec(memory_space=pltpu.VMEM),
        scratch_shapes=[
            pltpu.VMEM((half, D), jnp.float32),
            pltpu.SemaphoreType.DMA((4,)),
        ],
        compiler_params=pltpu.CompilerParams(collective_id=0),
    )(p2, resid, g2)


# device time: 38789 ns/iter; 1.5124x vs baseline; 1.5124x over previous
import jax
import jax.numpy as jnp
from jax import lax
from jax.experimental import pallas as pl
from jax.experimental.pallas import tpu as pltpu


def kernel(partial, resid, gamma):
    M, D = resid.shape
    half = M // 2
    p2 = partial.reshape(M, D)
    g2 = gamma.reshape(1, D)

    n_chunks = 8
    rows = half // n_chunks

    def body(p_ref, r_ref, g_ref, o_ref, yrecv_ref,
             ysend_sems, yrecv_sems, xsend_sems, xrecv_sems):
        my_x = lax.axis_index("x")
        my_y = lax.axis_index("y")
        row0 = my_x * half
        y_nbr = (my_x, 1 - my_y)
        x_nbr = (1 - my_x, my_y)

        barrier_sem = pltpu.get_barrier_semaphore()
        for nbr in (y_nbr, x_nbr):
            pl.semaphore_signal(
                barrier_sem, inc=1,
                device_id=nbr, device_id_type=pl.DeviceIdType.MESH,
            )
        pl.semaphore_wait(barrier_sem, 2)

        rdmas_y = []
        for c in range(n_chunks):
            rdma_y = pltpu.make_async_remote_copy(
                src_ref=p_ref.at[pl.ds(row0 + c * rows, rows), :],
                dst_ref=yrecv_ref.at[pl.ds(c * rows, rows), :],
                send_sem=ysend_sems.at[c],
                recv_sem=yrecv_sems.at[c],
                device_id=y_nbr,
                device_id_type=pl.DeviceIdType.MESH,
            )
            rdma_y.start()
            rdmas_y.append(rdma_y)

        rdmas_x = []
        for c in range(n_chunks):
            rdmas_y[c].wait_recv()
            yh = p_ref[pl.ds(row0 + c * rows, rows), :] \
                + yrecv_ref[pl.ds(c * rows, rows), :] \
                + r_ref[pl.ds(row0 + c * rows, rows), :]
            rms = jnp.sqrt(jnp.mean(yh * yh, axis=1, keepdims=True) + 1e-6)
            o_ref[pl.ds(row0 + c * rows, rows), :] = yh / rms * g_ref[:, :]

            rdma_x = pltpu.make_async_remote_copy(
                src_ref=o_ref.at[pl.ds(row0 + c * rows, rows), :],
                dst_ref=o_ref.at[pl.ds(row0 + c * rows, rows), :],
                send_sem=xsend_sems.at[c],
                recv_sem=xrecv_sems.at[c],
                device_id=x_nbr,
                device_id_type=pl.DeviceIdType.MESH,
            )
            rdma_x.start()
            rdmas_x.append(rdma_x)

        for c in range(n_chunks):
            rdmas_y[c].wait_send()
            rdmas_x[c].wait()

    return pl.pallas_call(
        body,
        out_shape=jax.ShapeDtypeStruct((M, D), jnp.float32),
        in_specs=[
            pl.BlockSpec(memory_space=pltpu.VMEM),
            pl.BlockSpec(memory_space=pltpu.VMEM),
            pl.BlockSpec(memory_space=pltpu.VMEM),
        ],
        out_specs=pl.BlockSpec(memory_space=pltpu.VMEM),
        scratch_shapes=[
            pltpu.VMEM((half, D), jnp.float32),
            pltpu.SemaphoreType.DMA((n_chunks,)),
            pltpu.SemaphoreType.DMA((n_chunks,)),
            pltpu.SemaphoreType.DMA((n_chunks,)),
            pltpu.SemaphoreType.DMA((n_chunks,)),
        ],
        compiler_params=pltpu.CompilerParams(collective_id=0),
    )(p2, resid, g2)


# device time: 37534 ns/iter; 1.5630x vs baseline; 1.0334x over previous
import jax
import jax.numpy as jnp
from jax import lax
from jax.experimental import pallas as pl
from jax.experimental.pallas import tpu as pltpu


def kernel(partial, resid, gamma):
    M, D = resid.shape
    half = M // 2
    p2 = partial.reshape(M, D)
    g2 = gamma.reshape(1, D)

    n_chunks = 16
    rows = half // n_chunks

    def body(p_ref, r_ref, g_ref, o_ref, yrecv_ref,
             ysend_sems, yrecv_sems, xsend_sems, xrecv_sems):
        my_x = lax.axis_index("x")
        my_y = lax.axis_index("y")
        row0 = my_x * half
        y_nbr = (my_x, 1 - my_y)
        x_nbr = (1 - my_x, my_y)

        barrier_sem = pltpu.get_barrier_semaphore()
        for nbr in (y_nbr, x_nbr):
            pl.semaphore_signal(
                barrier_sem, inc=1,
                device_id=nbr, device_id_type=pl.DeviceIdType.MESH,
            )
        pl.semaphore_wait(barrier_sem, 2)

        rdmas_y = []
        for c in range(n_chunks):
            rdma_y = pltpu.make_async_remote_copy(
                src_ref=p_ref.at[pl.ds(row0 + c * rows, rows), :],
                dst_ref=yrecv_ref.at[pl.ds(c * rows, rows), :],
                send_sem=ysend_sems.at[c],
                recv_sem=yrecv_sems.at[c],
                device_id=y_nbr,
                device_id_type=pl.DeviceIdType.MESH,
            )
            rdma_y.start()
            rdmas_y.append(rdma_y)

        rdmas_x = []
        for c in range(n_chunks):
            rdmas_y[c].wait_recv()
            yh = p_ref[pl.ds(row0 + c * rows, rows), :] \
                + yrecv_ref[pl.ds(c * rows, rows), :] \
                + r_ref[pl.ds(row0 + c * rows, rows), :]
            rms = jnp.sqrt(jnp.mean(yh * yh, axis=1, keepdims=True) + 1e-6)
            o_ref[pl.ds(row0 + c * rows, rows), :] = yh / rms * g_ref[:, :]

            rdma_x = pltpu.make_async_remote_copy(
                src_ref=o_ref.at[pl.ds(row0 + c * rows, rows), :],
                dst_ref=o_ref.at[pl.ds(row0 + c * rows, rows), :],
                send_sem=xsend_sems.at[c],
                recv_sem=xrecv_sems.at[c],
                device_id=x_nbr,
                device_id_type=pl.DeviceIdType.MESH,
            )
            rdma_x.start()
            rdmas_x.append(rdma_x)

        for c in range(n_chunks):
            rdmas_y[c].wait_send()
            rdmas_x[c].wait()

    return pl.pallas_call(
        body,
        out_shape=jax.ShapeDtypeStruct((M, D), jnp.float32),
        in_specs=[
            pl.BlockSpec(memory_space=pltpu.VMEM),
            pl.BlockSpec(memory_space=pltpu.VMEM),
            pl.BlockSpec(memory_space=pltpu.VMEM),
        ],
        out_specs=pl.BlockSpec(memory_space=pltpu.VMEM),
        scratch_shapes=[
            pltpu.VMEM((half, D), jnp.float32),
            pltpu.SemaphoreType.DMA((n_chunks,)),
            pltpu.SemaphoreType.DMA((n_chunks,)),
            pltpu.SemaphoreType.DMA((n_chunks,)),
            pltpu.SemaphoreType.DMA((n_chunks,)),
        ],
        compiler_params=pltpu.CompilerParams(collective_id=0),
    )(p2, resid, g2)
